# Initial kernel scaffold; baseline (speedup 1.0000x reference)
import functools

import jax
import jax.numpy as jnp
from jax import lax
from jax.experimental import pallas as pl
from jax.experimental.pallas import tpu as pltpu
from jax.experimental.pallas import tpu_sc as plsc

N = 10000
E = 320000
IN_DIM = 128
HEADS = 4
HEAD_DIM = 16
LATENT = 32
NEG_SLOPE = 0.2

TW = 80
AW = 16
NW = 32
NS = 16
C = 80
EPW = E // NW
NCHUNK = EPW // C
RPT = N // NS

_BLK = 1250
_GRID = N // _BLK


def _tc_matmul_body(x_ref, w_ref, t_ref, a_ref):
    t = jnp.dot(x_ref[...], w_ref[...], preferred_element_type=jnp.float32)
    t_ref[...] = t[:, :TW]
    a_ref[...] = t[:, TW:TW + AW]


def _tc_matmul(x, wcat):
    in_dim = x.shape[1]
    return pl.pallas_call(
        _tc_matmul_body,
        grid=(_GRID,),
        in_specs=[
            pl.BlockSpec((_BLK, in_dim), lambda i: (i, 0)),
            pl.BlockSpec((in_dim, TW + AW), lambda i: (0, 0)),
        ],
        out_specs=[
            pl.BlockSpec((_BLK, TW), lambda i: (i, 0)),
            pl.BlockSpec((_BLK, AW), lambda i: (i, 0)),
        ],
        out_shape=[
            jax.ShapeDtypeStruct((N, TW), jnp.float32),
            jax.ShapeDtypeStruct((N, AW), jnp.float32),
        ],
    )(x, wcat)


def _make_sc_pass(nh, head_of_vreg):
    mesh = plsc.VectorSubcoreMesh(core_axis_name="c", subcore_axis_name="s")

    @functools.partial(
        pl.kernel,
        out_type=jax.ShapeDtypeStruct((2, N, TW), jnp.float32),
        mesh=mesh,
        scratch_types=[
            pltpu.VMEM((C,), jnp.int32),
            pltpu.VMEM((C,), jnp.int32),
            pltpu.VMEM((C, TW), jnp.float32),
            pltpu.VMEM((C, AW), jnp.float32),
            pltpu.VMEM((16,), jnp.float32),
            pltpu.VMEM_SHARED((N, TW), jnp.float32),
            pltpu.SemaphoreType.DMA,
            pltpu.SemaphoreType.DMA,
        ],
    )
    def sc_pass(table_hbm, adst_hbm, src_hbm, dst_hbm, zero_hbm, out_hbm,
                sidx, didx, rows, arows, exbuf, acc, sem1, sem2):
        cid = lax.axis_index("c")
        sid = lax.axis_index("s")
        wid = sid * 2 + cid

        pltpu.sync_copy(zero_hbm.at[pl.ds(sid * RPT, RPT)],
                        acc.at[pl.ds(sid * RPT, RPT)])
        plsc.subcore_barrier()

        lanes = lax.iota(jnp.int32, shape=(16,), dimension=0)
        exmask = jnp.where(lanes < nh, 1.0, 0.0).astype(jnp.float32)

        def chunk_body(g, _):
            eb = pl.multiple_of(wid * EPW + g * C, 8)
            pltpu.sync_copy(src_hbm.at[pl.ds(eb, C)], sidx)
            pltpu.sync_copy(dst_hbm.at[pl.ds(eb, C)], didx)
            cp1 = pltpu.async_copy(table_hbm.at[sidx], rows, sem1)
            cp2 = pltpu.async_copy(adst_hbm.at[didx], arows, sem2)
            cp1.wait()
            cp2.wait()

            def row_body(r, _):
                e = rows[r, pl.ds(64, 16)] + arows[r, :]
                e = jnp.maximum(e, e * NEG_SLOPE)
                ex = jnp.exp(e) * exmask
                rows[r, pl.ds(64, 16)] = ex
                exbuf[...] = ex
                for k in range(4):
                    s = exbuf[head_of_vreg[k]]
                    rows[r, pl.ds(16 * k, 16)] = rows[r, pl.ds(16 * k, 16)] * s
                return 0

            lax.fori_loop(0, C, row_body, 0)
            pltpu.sync_copy(rows, acc.at[didx], add=True)
            return 0

        lax.fori_loop(0, NCHUNK, chunk_body, 0)
        plsc.subcore_barrier()
        pltpu.sync_copy(acc.at[pl.ds(sid * RPT, RPT)],
                        out_hbm.at[cid, pl.ds(sid * RPT, RPT)])

    return sc_pass


_sc_pass1 = _make_sc_pass(HEADS, (0, 1, 2, 3))
_sc_pass2 = _make_sc_pass(2, (0, 0, 1, 1))


def _tc_mid_body(a_ref, sel_ref, b_ref, w_ref, t_ref, ad_ref):
    acc = a_ref[0] + a_ref[1]
    recip = 1.0 / jnp.maximum(acc[:, 64:68], 1e-30)
    div = jnp.dot(recip, sel_ref[...], preferred_element_type=jnp.float32)
    o = acc[:, :64] * div + b_ref[...]
    h = jnp.where(o > 0, o, jnp.exp(jnp.minimum(o, 0.0)) - 1.0)
    t = jnp.dot(h, w_ref[...], preferred_element_type=jnp.float32)
    t_ref[...] = t[:, :TW]
    ad_ref[...] = t[:, TW:TW + AW]


def _tc_mid(accs, sel, b1row, wcat2):
    return pl.pallas_call(
        _tc_mid_body,
        grid=(_GRID,),
        in_specs=[
            pl.BlockSpec((2, _BLK, TW), lambda i: (0, i, 0)),
            pl.BlockSpec((HEADS, 64), lambda i: (0, 0)),
            pl.BlockSpec((1, 64), lambda i: (0, 0)),
            pl.BlockSpec((64, TW + AW), lambda i: (0, 0)),
        ],
        out_specs=[
            pl.BlockSpec((_BLK, TW), lambda i: (i, 0)),
            pl.BlockSpec((_BLK, AW), lambda i: (i, 0)),
        ],
        out_shape=[
            jax.ShapeDtypeStruct((N, TW), jnp.float32),
            jax.ShapeDtypeStruct((N, AW), jnp.float32),
        ],
    )(accs, sel, b1row, wcat2)


def _tc_fin_body(a_ref, bm_ref, bv_ref, mu_ref, lv_ref):
    acc = a_ref[0] + a_ref[1]
    dm = 1.0 / jnp.maximum(acc[:, 64:65], 1e-30)
    dv = 1.0 / jnp.maximum(acc[:, 65:66], 1e-30)
    mu_ref[...] = acc[:, 0:32] * dm + bm_ref[...]
    lv_ref[...] = acc[:, 32:64] * dv + bv_ref[...]


def _tc_fin(accs, bm, bv):
    return pl.pallas_call(
        _tc_fin_body,
        grid=(_GRID,),
        in_specs=[
            pl.BlockSpec((2, _BLK, TW), lambda i: (0, i, 0)),
            pl.BlockSpec((1, LATENT), lambda i: (0, 0)),
            pl.BlockSpec((1, LATENT), lambda i: (0, 0)),
        ],
        out_specs=[
            pl.BlockSpec((_BLK, LATENT), lambda i: (i, 0)),
            pl.BlockSpec((_BLK, LATENT), lambda i: (i, 0)),
        ],
        out_shape=[
            jax.ShapeDtypeStruct((N, LATENT), jnp.float32),
            jax.ShapeDtypeStruct((N, LATENT), jnp.float32),
        ],
    )(accs, bm, bv)


def kernel(x, edge_index, W1, a_src1, a_dst1, b1,
           Wm, a_srcm, a_dstm, bm, Wv, a_srcv, a_dstv, bv):
    f32 = jnp.float32
    src = edge_index[0]
    dst = edge_index[1]
    zeros = jnp.zeros((N, TW), f32)

    j = jnp.arange(HEADS * HEAD_DIM)
    selT = (j[:, None] // HEAD_DIM == jnp.arange(HEADS)[None, :]).astype(f32)
    A_src = a_src1.reshape(-1)[:, None] * selT
    A_dst = a_dst1.reshape(-1)[:, None] * selT
    z12 = jnp.zeros((IN_DIM, 12), f32)
    wcat1 = jnp.concatenate(
        [W1, W1 @ A_src, z12, W1 @ A_dst, z12], axis=1)

    sel = selT.T
    b1row = b1.reshape(1, 64)
    z14 = jnp.zeros((64, 14), f32)
    wcat2 = jnp.concatenate(
        [Wm, Wv, Wm @ a_srcm.T, Wv @ a_srcv.T, z14,
         Wm @ a_dstm.T, Wv @ a_dstv.T, z14], axis=1)

    table1, adst1 = _tc_matmul(x, wcat1)
    accs1 = _sc_pass1(table1, adst1, src, dst, zeros)
    table2, adst2 = _tc_mid(accs1, sel, b1row, wcat2)
    accs2 = _sc_pass2(table2, adst2, src, dst, zeros)
    mu, logvar = _tc_fin(accs2, bm.reshape(1, LATENT), bv.reshape(1, LATENT))
    return (mu, logvar)


# speedup vs baseline: 44.9147x; 44.9147x over previous
import functools

import jax
import jax.numpy as jnp
from jax import lax
from jax.experimental import pallas as pl
from jax.experimental.pallas import tpu as pltpu
from jax.experimental.pallas import tpu_sc as plsc

N = 10000
E = 320000
IN_DIM = 128
HEADS = 4
HEAD_DIM = 16
LATENT = 32
NEG_SLOPE = 0.2

TW = 80
AW = 16
NW = 32
NS = 16
C = 80
EPW = E // NW
NCHUNK = EPW // C
RPT = 624
REM = N - NS * RPT

_BLK = 2000
_GRID = N // _BLK


def _tc_matmul_body(x_ref, w_ref, t_ref, a_ref):
    t = jnp.dot(x_ref[...], w_ref[...], preferred_element_type=jnp.float32)
    t_ref[...] = t[:, :TW]
    a_ref[...] = t[:, TW:TW + AW]


def _tc_matmul(x, wcat):
    in_dim = x.shape[1]
    return pl.pallas_call(
        _tc_matmul_body,
        grid=(_GRID,),
        in_specs=[
            pl.BlockSpec((_BLK, in_dim), lambda i: (i, 0)),
            pl.BlockSpec((in_dim, TW + AW), lambda i: (0, 0)),
        ],
        out_specs=[
            pl.BlockSpec((_BLK, TW), lambda i: (i, 0)),
            pl.BlockSpec((_BLK, AW), lambda i: (i, 0)),
        ],
        out_shape=[
            jax.ShapeDtypeStruct((N, TW), jnp.float32),
            jax.ShapeDtypeStruct((N, AW), jnp.float32),
        ],
    )(x, wcat)


def _make_sc_pass(nh, head_of_vreg):
    mesh = plsc.VectorSubcoreMesh(core_axis_name="c", subcore_axis_name="s")

    @functools.partial(
        pl.kernel,
        out_type=jax.ShapeDtypeStruct((2, N, TW), jnp.float32),
        mesh=mesh,
        scratch_types=[
            pltpu.VMEM((C,), jnp.int32),
            pltpu.VMEM((C,), jnp.int32),
            pltpu.VMEM((C, TW), jnp.float32),
            pltpu.VMEM((C, AW), jnp.float32),
            pltpu.VMEM_SHARED((N, TW), jnp.float32),
            pltpu.SemaphoreType.DMA,
            pltpu.SemaphoreType.DMA,
        ],
        compiler_params=pltpu.CompilerParams(use_tc_tiling_on_sc=False),
    )
    def sc_pass(table_hbm, adst_hbm, src_hbm, dst_hbm, zero_hbm, out_hbm,
                sidx, didx, rows, arows, acc, sem1, sem2):
        cid = lax.axis_index("c")
        sid = lax.axis_index("s")
        wid = sid * 2 + cid

        pltpu.sync_copy(zero_hbm.at[pl.ds(sid * RPT, RPT)],
                        acc.at[pl.ds(sid * RPT, RPT)])

        @pl.when(sid == NS - 1)
        def _():
            pltpu.sync_copy(zero_hbm.at[pl.ds(NS * RPT, REM)],
                            acc.at[pl.ds(NS * RPT, REM)])

        plsc.subcore_barrier()

        lanes = lax.iota(jnp.int32, 16)
        exmask = jnp.where(lanes < nh, 1.0, 0.0).astype(jnp.float32)

        def chunk_body(g, _):
            eb = pl.multiple_of(wid * EPW + g * C, 8)
            pltpu.sync_copy(src_hbm.at[pl.ds(eb, C)], sidx)
            pltpu.sync_copy(dst_hbm.at[pl.ds(eb, C)], didx)
            cp1 = pltpu.async_copy(table_hbm.at[sidx], rows, sem1)
            cp2 = pltpu.async_copy(adst_hbm.at[didx], arows, sem2)
            cp1.wait()
            cp2.wait()

            def row_body(r, _):
                e = rows[r, pl.ds(64, 16)] + arows[r, :]
                e = jnp.maximum(e, e * NEG_SLOPE)
                ex = jnp.exp(e) * exmask
                rows[r, pl.ds(64, 16)] = ex
                for k in range(4):
                    s = ex[head_of_vreg[k]]
                    rows[r, pl.ds(16 * k, 16)] = rows[r, pl.ds(16 * k, 16)] * s
                return 0

            lax.fori_loop(0, C, row_body, 0)
            pltpu.sync_copy(rows, acc.at[didx], add=True)
            return 0

        lax.fori_loop(0, NCHUNK, chunk_body, 0)
        plsc.subcore_barrier()
        pltpu.sync_copy(acc.at[pl.ds(sid * RPT, RPT)],
                        out_hbm.at[cid, pl.ds(sid * RPT, RPT)])

        @pl.when(sid == NS - 1)
        def _():
            pltpu.sync_copy(acc.at[pl.ds(NS * RPT, REM)],
                            out_hbm.at[cid, pl.ds(NS * RPT, REM)])

    return sc_pass


_sc_pass1 = _make_sc_pass(HEADS, (0, 1, 2, 3))
_sc_pass2 = _make_sc_pass(2, (0, 0, 1, 1))


def _tc_mid_body(a_ref, sel_ref, b_ref, w_ref, t_ref, ad_ref):
    acc = a_ref[0] + a_ref[1]
    recip = 1.0 / jnp.maximum(acc[:, 64:68], 1e-30)
    div = jnp.dot(recip, sel_ref[...], preferred_element_type=jnp.float32)
    o = acc[:, :64] * div + b_ref[...]
    h = jnp.where(o > 0, o, jnp.exp(jnp.minimum(o, 0.0)) - 1.0)
    t = jnp.dot(h, w_ref[...], preferred_element_type=jnp.float32)
    t_ref[...] = t[:, :TW]
    ad_ref[...] = t[:, TW:TW + AW]


def _tc_mid(accs, sel, b1row, wcat2):
    return pl.pallas_call(
        _tc_mid_body,
        grid=(_GRID,),
        in_specs=[
            pl.BlockSpec((2, _BLK, TW), lambda i: (0, i, 0)),
            pl.BlockSpec((HEADS, 64), lambda i: (0, 0)),
            pl.BlockSpec((1, 64), lambda i: (0, 0)),
            pl.BlockSpec((64, TW + AW), lambda i: (0, 0)),
        ],
        out_specs=[
            pl.BlockSpec((_BLK, TW), lambda i: (i, 0)),
            pl.BlockSpec((_BLK, AW), lambda i: (i, 0)),
        ],
        out_shape=[
            jax.ShapeDtypeStruct((N, TW), jnp.float32),
            jax.ShapeDtypeStruct((N, AW), jnp.float32),
        ],
    )(accs, sel, b1row, wcat2)


def _tc_fin_body(a_ref, bm_ref, bv_ref, mu_ref, lv_ref):
    acc = a_ref[0] + a_ref[1]
    dm = 1.0 / jnp.maximum(acc[:, 64:65], 1e-30)
    dv = 1.0 / jnp.maximum(acc[:, 65:66], 1e-30)
    mu_ref[...] = acc[:, 0:32] * dm + bm_ref[...]
    lv_ref[...] = acc[:, 32:64] * dv + bv_ref[...]


def _tc_fin(accs, bm, bv):
    return pl.pallas_call(
        _tc_fin_body,
        grid=(_GRID,),
        in_specs=[
            pl.BlockSpec((2, _BLK, TW), lambda i: (0, i, 0)),
            pl.BlockSpec((1, LATENT), lambda i: (0, 0)),
            pl.BlockSpec((1, LATENT), lambda i: (0, 0)),
        ],
        out_specs=[
            pl.BlockSpec((_BLK, LATENT), lambda i: (i, 0)),
            pl.BlockSpec((_BLK, LATENT), lambda i: (i, 0)),
        ],
        out_shape=[
            jax.ShapeDtypeStruct((N, LATENT), jnp.float32),
            jax.ShapeDtypeStruct((N, LATENT), jnp.float32),
        ],
    )(accs, bm, bv)


def kernel(x, edge_index, W1, a_src1, a_dst1, b1,
           Wm, a_srcm, a_dstm, bm, Wv, a_srcv, a_dstv, bv):
    f32 = jnp.float32
    src = edge_index[0]
    dst = edge_index[1]
    zeros = jnp.zeros((N, TW), f32)

    j = jnp.arange(HEADS * HEAD_DIM)
    selT = (j[:, None] // HEAD_DIM == jnp.arange(HEADS)[None, :]).astype(f32)
    A_src = a_src1.reshape(-1)[:, None] * selT
    A_dst = a_dst1.reshape(-1)[:, None] * selT
    z12 = jnp.zeros((IN_DIM, 12), f32)
    wcat1 = jnp.concatenate(
        [W1, W1 @ A_src, z12, W1 @ A_dst, z12], axis=1)

    sel = selT.T
    b1row = b1.reshape(1, 64)
    z14 = jnp.zeros((64, 14), f32)
    wcat2 = jnp.concatenate(
        [Wm, Wv, Wm @ a_srcm.T, Wv @ a_srcv.T, z14,
         Wm @ a_dstm.T, Wv @ a_dstv.T, z14], axis=1)

    table1, adst1 = _tc_matmul(x, wcat1)
    accs1 = _sc_pass1(table1, adst1, src, dst, zeros)
    table2, adst2 = _tc_mid(accs1, sel, b1row, wcat2)
    accs2 = _sc_pass2(table2, adst2, src, dst, zeros)
    mu, logvar = _tc_fin(accs2, bm.reshape(1, LATENT), bv.reshape(1, LATENT))
    return (mu, logvar)


# speedup vs baseline: 91.7779x; 2.0434x over previous
import functools

import jax
import jax.numpy as jnp
from jax import lax
from jax.experimental import pallas as pl
from jax.experimental.pallas import tpu as pltpu
from jax.experimental.pallas import tpu_sc as plsc

N = 10000
E = 320000
IN_DIM = 128
HEADS = 4
HEAD_DIM = 16
LATENT = 32
NEG_SLOPE = 0.2

TW = 80
AW = 16
NW = 32
NS = 16
C = 80
EPW = E // NW
NCHUNK = EPW // C
RPT = 624
REM = N - NS * RPT

_BLK = 2000
_GRID = N // _BLK


def _tc_matmul_body(x_ref, w_ref, t_ref, a_ref):
    t = jnp.dot(x_ref[...], w_ref[...], preferred_element_type=jnp.float32)
    t_ref[...] = t[:, :TW]
    a_ref[...] = t[:, TW:TW + AW]


def _tc_matmul(x, wcat):
    in_dim = x.shape[1]
    return pl.pallas_call(
        _tc_matmul_body,
        grid=(_GRID,),
        in_specs=[
            pl.BlockSpec((_BLK, in_dim), lambda i: (i, 0)),
            pl.BlockSpec((in_dim, TW + AW), lambda i: (0, 0)),
        ],
        out_specs=[
            pl.BlockSpec((_BLK, TW), lambda i: (i, 0)),
            pl.BlockSpec((_BLK, AW), lambda i: (i, 0)),
        ],
        out_shape=[
            jax.ShapeDtypeStruct((N, TW), jnp.float32),
            jax.ShapeDtypeStruct((N, AW), jnp.float32),
        ],
    )(x, wcat)


def _make_sc_pass(nh, head_of_vreg):
    mesh = plsc.VectorSubcoreMesh(core_axis_name="c", subcore_axis_name="s")

    @functools.partial(
        pl.kernel,
        out_type=jax.ShapeDtypeStruct((2, N, TW), jnp.float32),
        mesh=mesh,
        scratch_types=[
            [pltpu.VMEM((C,), jnp.int32)] * 2,
            [pltpu.VMEM((C,), jnp.int32)] * 2,
            [pltpu.VMEM((C,), jnp.int32)] * 2,
            [pltpu.VMEM((C, TW), jnp.float32)] * 2,
            [pltpu.VMEM((C, AW), jnp.float32)] * 2,
            [pltpu.VMEM((C, TW), jnp.float32)] * 2,
            [pltpu.SemaphoreType.DMA] * 2,
            [pltpu.SemaphoreType.DMA] * 2,
            [pltpu.SemaphoreType.DMA] * 2,
            [pltpu.SemaphoreType.DMA] * 2,
            pltpu.VMEM_SHARED((N, TW), jnp.float32),
        ],
        compiler_params=pltpu.CompilerParams(use_tc_tiling_on_sc=False),
    )
    def sc_pass(table_hbm, adst_hbm, src_hbm, dst_hbm, zero_hbm, out_hbm,
                sidx, didx, sdix, rows, arows, outb,
                gsem_t, gsem_a, isem, ssem, acc):
        cid = lax.axis_index("c")
        sid = lax.axis_index("s")
        wid = sid * 2 + cid

        pltpu.sync_copy(zero_hbm.at[pl.ds(sid * RPT, RPT)],
                        acc.at[pl.ds(sid * RPT, RPT)])

        @pl.when(sid == NS - 1)
        def _():
            pltpu.sync_copy(zero_hbm.at[pl.ds(NS * RPT, REM)],
                            acc.at[pl.ds(NS * RPT, REM)])

        plsc.subcore_barrier()

        lanes = lax.iota(jnp.int32, 16)
        exmask = jnp.where(lanes < nh, 1.0, 0.0).astype(jnp.float32)

        def ebase(g):
            return pl.multiple_of(wid * EPW + g * C, 8)

        def issue_idx(g, b):
            eb = ebase(g)
            pltpu.async_copy(src_hbm.at[pl.ds(eb, C)], sidx[b], isem[b])
            pltpu.async_copy(dst_hbm.at[pl.ds(eb, C)], didx[b], isem[b])

        def wait_idx(b):
            pltpu.make_async_copy(src_hbm.at[pl.ds(0, C)], sidx[b],
                                  isem[b]).wait()
            pltpu.make_async_copy(dst_hbm.at[pl.ds(0, C)], didx[b],
                                  isem[b]).wait()

        def issue_gather(b):
            pltpu.async_copy(table_hbm.at[sidx[b]], rows[b], gsem_t[b])
            pltpu.async_copy(adst_hbm.at[didx[b]], arows[b], gsem_a[b])

        def wait_gather(b):
            pltpu.make_async_copy(table_hbm.at[sidx[b]], rows[b],
                                  gsem_t[b]).wait()
            pltpu.make_async_copy(adst_hbm.at[didx[b]], arows[b],
                                  gsem_a[b]).wait()

        def wait_scatter(b):
            pltpu.make_async_copy(outb[b], acc.at[sdix[b]], ssem[b]).wait()

        def compute(b):
            rb, ab, ob = rows[b], arows[b], outb[b]

            def row_body(r, _):
                e = rb[r, pl.ds(64, 16)] + ab[r, :]
                e = jnp.maximum(e, e * NEG_SLOPE)
                ex = jnp.exp(e) * exmask
                ob[r, pl.ds(64, 16)] = ex
                for k in range(4):
                    s = ex[head_of_vreg[k]]
                    ob[r, pl.ds(16 * k, 16)] = rb[r, pl.ds(16 * k, 16)] * s
                return 0

            lax.fori_loop(0, C, row_body, 0)

        def chunk(g, b, first_pair):
            wait_gather(b)

            @pl.when(jnp.logical_not(first_pair))
            def _():
                wait_scatter(b)
            for v in range(C // 16):
                sdix[b][pl.ds(16 * v, 16)] = didx[b][pl.ds(16 * v, 16)]

            @pl.when(g + 2 < NCHUNK)
            def _():
                issue_idx(g + 2, b)
            compute(b)
            pltpu.async_copy(outb[b], acc.at[sdix[b]], ssem[b], add=True)

            @pl.when(g + 2 < NCHUNK)
            def _():
                wait_idx(b)
                issue_gather(b)

        for b in (0, 1):
            issue_idx(b, b)
            wait_idx(b)
            issue_gather(b)

        def pair_body(p, _):
            g = p * 2
            chunk(g, 0, p == 0)
            chunk(g + 1, 1, p == 0)
            return 0

        lax.fori_loop(0, NCHUNK // 2, pair_body, 0)
        chunk(NCHUNK - 1, 0, False)
        wait_scatter(0)
        wait_scatter(1)

        plsc.subcore_barrier()
        pltpu.sync_copy(acc.at[pl.ds(sid * RPT, RPT)],
                        out_hbm.at[cid, pl.ds(sid * RPT, RPT)])

        @pl.when(sid == NS - 1)
        def _():
            pltpu.sync_copy(acc.at[pl.ds(NS * RPT, REM)],
                            out_hbm.at[cid, pl.ds(NS * RPT, REM)])

    return sc_pass


_sc_pass1 = _make_sc_pass(HEADS, (0, 1, 2, 3))
_sc_pass2 = _make_sc_pass(2, (0, 0, 1, 1))


def _tc_mid_body(a_ref, sel_ref, b_ref, w_ref, t_ref, ad_ref):
    acc = a_ref[0] + a_ref[1]
    recip = 1.0 / jnp.maximum(acc[:, 64:68], 1e-30)
    div = jnp.dot(recip, sel_ref[...], preferred_element_type=jnp.float32)
    o = acc[:, :64] * div + b_ref[...]
    h = jnp.where(o > 0, o, jnp.exp(jnp.minimum(o, 0.0)) - 1.0)
    t = jnp.dot(h, w_ref[...], preferred_element_type=jnp.float32)
    t_ref[...] = t[:, :TW]
    ad_ref[...] = t[:, TW:TW + AW]


def _tc_mid(accs, sel, b1row, wcat2):
    return pl.pallas_call(
        _tc_mid_body,
        grid=(_GRID,),
        in_specs=[
            pl.BlockSpec((2, _BLK, TW), lambda i: (0, i, 0)),
            pl.BlockSpec((HEADS, 64), lambda i: (0, 0)),
            pl.BlockSpec((1, 64), lambda i: (0, 0)),
            pl.BlockSpec((64, TW + AW), lambda i: (0, 0)),
        ],
        out_specs=[
            pl.BlockSpec((_BLK, TW), lambda i: (i, 0)),
            pl.BlockSpec((_BLK, AW), lambda i: (i, 0)),
        ],
        out_shape=[
            jax.ShapeDtypeStruct((N, TW), jnp.float32),
            jax.ShapeDtypeStruct((N, AW), jnp.float32),
        ],
    )(accs, sel, b1row, wcat2)


def _tc_fin_body(a_ref, bm_ref, bv_ref, mu_ref, lv_ref):
    acc = a_ref[0] + a_ref[1]
    dm = 1.0 / jnp.maximum(acc[:, 64:65], 1e-30)
    dv = 1.0 / jnp.maximum(acc[:, 65:66], 1e-30)
    mu_ref[...] = acc[:, 0:32] * dm + bm_ref[...]
    lv_ref[...] = acc[:, 32:64] * dv + bv_ref[...]


def _tc_fin(accs, bm, bv):
    return pl.pallas_call(
        _tc_fin_body,
        grid=(_GRID,),
        in_specs=[
            pl.BlockSpec((2, _BLK, TW), lambda i: (0, i, 0)),
            pl.BlockSpec((1, LATENT), lambda i: (0, 0)),
            pl.BlockSpec((1, LATENT), lambda i: (0, 0)),
        ],
        out_specs=[
            pl.BlockSpec((_BLK, LATENT), lambda i: (i, 0)),
            pl.BlockSpec((_BLK, LATENT), lambda i: (i, 0)),
        ],
        out_shape=[
            jax.ShapeDtypeStruct((N, LATENT), jnp.float32),
            jax.ShapeDtypeStruct((N, LATENT), jnp.float32),
        ],
    )(accs, bm, bv)


def kernel(x, edge_index, W1, a_src1, a_dst1, b1,
           Wm, a_srcm, a_dstm, bm, Wv, a_srcv, a_dstv, bv):
    f32 = jnp.float32
    src = edge_index[0]
    dst = edge_index[1]
    zeros = jnp.zeros((N, TW), f32)

    j = jnp.arange(HEADS * HEAD_DIM)
    selT = (j[:, None] // HEAD_DIM == jnp.arange(HEADS)[None, :]).astype(f32)
    A_src = a_src1.reshape(-1)[:, None] * selT
    A_dst = a_dst1.reshape(-1)[:, None] * selT
    z12 = jnp.zeros((IN_DIM, 12), f32)
    wcat1 = jnp.concatenate(
        [W1, W1 @ A_src, z12, W1 @ A_dst, z12], axis=1)

    sel = selT.T
    b1row = b1.reshape(1, 64)
    z14 = jnp.zeros((64, 14), f32)
    wcat2 = jnp.concatenate(
        [Wm, Wv, Wm @ a_srcm.T, Wv @ a_srcv.T, z14,
         Wm @ a_dstm.T, Wv @ a_dstv.T, z14], axis=1)

    table1, adst1 = _tc_matmul(x, wcat1)
    accs1 = _sc_pass1(table1, adst1, src, dst, zeros)
    table2, adst2 = _tc_mid(accs1, sel, b1row, wcat2)
    accs2 = _sc_pass2(table2, adst2, src, dst, zeros)
    mu, logvar = _tc_fin(accs2, bm.reshape(1, LATENT), bv.reshape(1, LATENT))
    return (mu, logvar)


# speedup vs baseline: 145.8717x; 1.5894x over previous
import functools

import jax
import jax.numpy as jnp
from jax import lax
from jax.experimental import pallas as pl
from jax.experimental.pallas import tpu as pltpu
from jax.experimental.pallas import tpu_sc as plsc

N = 10000
E = 320000
IN_DIM = 128
HEADS = 4
HEAD_DIM = 16
LATENT = 32
NEG_SLOPE = 0.2

TW = 80
AW = 16
NW = 32
NS = 16
C = 80
EPW = E // NW
NCHUNK = EPW // C
RPT = 624
REM = N - NS * RPT

_BLK = 2000
_GRID = N // _BLK


def _tc_matmul_body(x_ref, w_ref, t_ref, a_ref):
    t = jnp.dot(x_ref[...], w_ref[...], preferred_element_type=jnp.float32)
    t_ref[...] = t[:, :TW]
    a_ref[...] = t[:, TW:TW + AW]


def _tc_matmul(x, wcat):
    in_dim = x.shape[1]
    return pl.pallas_call(
        _tc_matmul_body,
        grid=(_GRID,),
        in_specs=[
            pl.BlockSpec((_BLK, in_dim), lambda i: (i, 0)),
            pl.BlockSpec((in_dim, TW + AW), lambda i: (0, 0)),
        ],
        out_specs=[
            pl.BlockSpec((_BLK, TW), lambda i: (i, 0)),
            pl.BlockSpec((_BLK, AW), lambda i: (i, 0)),
        ],
        out_shape=[
            jax.ShapeDtypeStruct((N, TW), jnp.float32),
            jax.ShapeDtypeStruct((N, AW), jnp.float32),
        ],
    )(x, wcat)


def _make_sc_pass(nh, head_of_vreg):
    mesh = plsc.VectorSubcoreMesh(core_axis_name="c", subcore_axis_name="s")

    @functools.partial(
        pl.kernel,
        out_type=jax.ShapeDtypeStruct((2, N, TW), jnp.float32),
        mesh=mesh,
        scratch_types=[
            [pltpu.VMEM((C,), jnp.int32)] * 2,
            [pltpu.VMEM((C,), jnp.int32)] * 2,
            [pltpu.VMEM((C,), jnp.int32)] * 2,
            [pltpu.VMEM((C, TW), jnp.float32)] * 2,
            [pltpu.VMEM((C, AW), jnp.float32)] * 2,
            [pltpu.VMEM((C, TW), jnp.float32)] * 2,
            [pltpu.SemaphoreType.DMA] * 2,
            [pltpu.SemaphoreType.DMA] * 2,
            [pltpu.SemaphoreType.DMA] * 2,
            [pltpu.SemaphoreType.DMA] * 2,
            pltpu.VMEM_SHARED((N, TW), jnp.float32),
        ],
        compiler_params=pltpu.CompilerParams(use_tc_tiling_on_sc=False),
    )
    def sc_pass(table_hbm, adst_hbm, src_hbm, dst_hbm, zero_hbm, out_hbm,
                sidx, didx, sdix, rows, arows, outb,
                gsem_t, gsem_a, isem, ssem, acc):
        cid = lax.axis_index("c")
        sid = lax.axis_index("s")
        wid = sid * 2 + cid

        pltpu.sync_copy(zero_hbm.at[pl.ds(sid * RPT, RPT)],
                        acc.at[pl.ds(sid * RPT, RPT)])

        @pl.when(sid == NS - 1)
        def _():
            pltpu.sync_copy(zero_hbm.at[pl.ds(NS * RPT, REM)],
                            acc.at[pl.ds(NS * RPT, REM)])

        plsc.subcore_barrier()

        lanes = lax.iota(jnp.int32, 16)
        exmask = jnp.where(lanes < nh, 1.0, 0.0).astype(jnp.float32)

        def ebase(g):
            return pl.multiple_of(wid * EPW + g * C, 8)

        def issue_idx(g, b):
            eb = ebase(g)
            pltpu.async_copy(src_hbm.at[pl.ds(eb, C)], sidx[b], isem[b])
            pltpu.async_copy(dst_hbm.at[pl.ds(eb, C)], didx[b], isem[b])

        def wait_idx(b):
            pltpu.make_async_copy(src_hbm.at[pl.ds(0, C)], sidx[b],
                                  isem[b]).wait()
            pltpu.make_async_copy(dst_hbm.at[pl.ds(0, C)], didx[b],
                                  isem[b]).wait()

        def issue_gather(b):
            pltpu.async_copy(table_hbm.at[sidx[b]], rows[b], gsem_t[b])
            pltpu.async_copy(adst_hbm.at[didx[b]], arows[b], gsem_a[b])

        def wait_gather(b):
            pltpu.make_async_copy(table_hbm.at[sidx[b]], rows[b],
                                  gsem_t[b]).wait()
            pltpu.make_async_copy(adst_hbm.at[didx[b]], arows[b],
                                  gsem_a[b]).wait()

        def wait_scatter(b):
            pltpu.make_async_copy(outb[b], acc.at[sdix[b]], ssem[b]).wait()

        def compute(b):
            rb, ab, ob = rows[b], arows[b], outb[b]

            @plsc.parallel_loop(0, C, 1, unroll=4)
            def _(r):
                e = rb[r, pl.ds(64, 16)] + ab[r, :]
                e = jnp.maximum(e, e * NEG_SLOPE)
                ex = jnp.exp(e) * exmask
                ob[r, pl.ds(64, 16)] = ex
                for k in range(4):
                    s = ex[head_of_vreg[k]]
                    ob[r, pl.ds(16 * k, 16)] = rb[r, pl.ds(16 * k, 16)] * s

        def chunk(g, b, first_pair):
            wait_gather(b)

            @pl.when(jnp.logical_not(first_pair))
            def _():
                wait_scatter(b)
            for v in range(C // 16):
                sdix[b][pl.ds(16 * v, 16)] = didx[b][pl.ds(16 * v, 16)]

            @pl.when(g + 2 < NCHUNK)
            def _():
                issue_idx(g + 2, b)
            compute(b)
            pltpu.async_copy(outb[b], acc.at[sdix[b]], ssem[b], add=True)

            @pl.when(g + 2 < NCHUNK)
            def _():
                wait_idx(b)
                issue_gather(b)

        for b in (0, 1):
            issue_idx(b, b)
            wait_idx(b)
            issue_gather(b)

        def pair_body(p, _):
            g = p * 2
            chunk(g, 0, p == 0)
            chunk(g + 1, 1, p == 0)
            return 0

        lax.fori_loop(0, NCHUNK // 2, pair_body, 0)
        chunk(NCHUNK - 1, 0, False)
        wait_scatter(0)
        wait_scatter(1)

        plsc.subcore_barrier()
        pltpu.sync_copy(acc.at[pl.ds(sid * RPT, RPT)],
                        out_hbm.at[cid, pl.ds(sid * RPT, RPT)])

        @pl.when(sid == NS - 1)
        def _():
            pltpu.sync_copy(acc.at[pl.ds(NS * RPT, REM)],
                            out_hbm.at[cid, pl.ds(NS * RPT, REM)])

    return sc_pass


_sc_pass1 = _make_sc_pass(HEADS, (0, 1, 2, 3))
_sc_pass2 = _make_sc_pass(2, (0, 0, 1, 1))


def _tc_mid_body(a_ref, sel_ref, b_ref, w_ref, t_ref, ad_ref):
    acc = a_ref[0] + a_ref[1]
    recip = 1.0 / jnp.maximum(acc[:, 64:68], 1e-30)
    div = jnp.dot(recip, sel_ref[...], preferred_element_type=jnp.float32)
    o = acc[:, :64] * div + b_ref[...]
    h = jnp.where(o > 0, o, jnp.exp(jnp.minimum(o, 0.0)) - 1.0)
    t = jnp.dot(h, w_ref[...], preferred_element_type=jnp.float32)
    t_ref[...] = t[:, :TW]
    ad_ref[...] = t[:, TW:TW + AW]


def _tc_mid(accs, sel, b1row, wcat2):
    return pl.pallas_call(
        _tc_mid_body,
        grid=(_GRID,),
        in_specs=[
            pl.BlockSpec((2, _BLK, TW), lambda i: (0, i, 0)),
            pl.BlockSpec((HEADS, 64), lambda i: (0, 0)),
            pl.BlockSpec((1, 64), lambda i: (0, 0)),
            pl.BlockSpec((64, TW + AW), lambda i: (0, 0)),
        ],
        out_specs=[
            pl.BlockSpec((_BLK, TW), lambda i: (i, 0)),
            pl.BlockSpec((_BLK, AW), lambda i: (i, 0)),
        ],
        out_shape=[
            jax.ShapeDtypeStruct((N, TW), jnp.float32),
            jax.ShapeDtypeStruct((N, AW), jnp.float32),
        ],
    )(accs, sel, b1row, wcat2)


def _tc_fin_body(a_ref, bm_ref, bv_ref, mu_ref, lv_ref):
    acc = a_ref[0] + a_ref[1]
    dm = 1.0 / jnp.maximum(acc[:, 64:65], 1e-30)
    dv = 1.0 / jnp.maximum(acc[:, 65:66], 1e-30)
    mu_ref[...] = acc[:, 0:32] * dm + bm_ref[...]
    lv_ref[...] = acc[:, 32:64] * dv + bv_ref[...]


def _tc_fin(accs, bm, bv):
    return pl.pallas_call(
        _tc_fin_body,
        grid=(_GRID,),
        in_specs=[
            pl.BlockSpec((2, _BLK, TW), lambda i: (0, i, 0)),
            pl.BlockSpec((1, LATENT), lambda i: (0, 0)),
            pl.BlockSpec((1, LATENT), lambda i: (0, 0)),
        ],
        out_specs=[
            pl.BlockSpec((_BLK, LATENT), lambda i: (i, 0)),
            pl.BlockSpec((_BLK, LATENT), lambda i: (i, 0)),
        ],
        out_shape=[
            jax.ShapeDtypeStruct((N, LATENT), jnp.float32),
            jax.ShapeDtypeStruct((N, LATENT), jnp.float32),
        ],
    )(accs, bm, bv)


def kernel(x, edge_index, W1, a_src1, a_dst1, b1,
           Wm, a_srcm, a_dstm, bm, Wv, a_srcv, a_dstv, bv):
    f32 = jnp.float32
    src = edge_index[0]
    dst = edge_index[1]
    zeros = jnp.zeros((N, TW), f32)

    j = jnp.arange(HEADS * HEAD_DIM)
    selT = (j[:, None] // HEAD_DIM == jnp.arange(HEADS)[None, :]).astype(f32)
    A_src = a_src1.reshape(-1)[:, None] * selT
    A_dst = a_dst1.reshape(-1)[:, None] * selT
    z12 = jnp.zeros((IN_DIM, 12), f32)
    wcat1 = jnp.concatenate(
        [W1, W1 @ A_src, z12, W1 @ A_dst, z12], axis=1)

    sel = selT.T
    b1row = b1.reshape(1, 64)
    z14 = jnp.zeros((64, 14), f32)
    wcat2 = jnp.concatenate(
        [Wm, Wv, Wm @ a_srcm.T, Wv @ a_srcv.T, z14,
         Wm @ a_dstm.T, Wv @ a_dstv.T, z14], axis=1)

    table1, adst1 = _tc_matmul(x, wcat1)
    accs1 = _sc_pass1(table1, adst1, src, dst, zeros)
    table2, adst2 = _tc_mid(accs1, sel, b1row, wcat2)
    accs2 = _sc_pass2(table2, adst2, src, dst, zeros)
    mu, logvar = _tc_fin(accs2, bm.reshape(1, LATENT), bv.reshape(1, LATENT))
    return (mu, logvar)
